# baseline (speedup 1.0000x reference)
import functools

import jax
import jax.numpy as jnp
from jax.experimental import pallas as pl
from jax.experimental.pallas import tpu as pltpu

T = 2048
D = 2048
H = 16
HD = 128
TOPK = 1024
BT = 256
SC = 256
MASK_BASE = -1900000000


def _bf(a):
    return a.astype(jnp.bfloat16)


def _proj_kernel(x_ref, wk_ref, ww_ref, wqd_ref, k_ref, w_ref, qc_ref):
    x = _bf(x_ref[...])
    k_ref[...] = jnp.dot(x, _bf(wk_ref[...]),
                         preferred_element_type=jnp.float32)
    w_ref[...] = jnp.dot(x, _bf(ww_ref[...]),
                         preferred_element_type=jnp.float32)
    qc_ref[...] = jnp.dot(x, _bf(wqd_ref[...]),
                          preferred_element_type=jnp.float32)


def _scores_kernel(qc_ref, wqu_ref, k_ref, w_ref, s_ref):
    i = pl.program_id(0)
    q = jnp.dot(_bf(qc_ref[...]), _bf(wqu_ref[...]),
                preferred_element_type=jnp.float32)
    k = _bf(k_ref[...])
    w = w_ref[...]
    terms = []
    for h in range(H):
        qh = _bf(q[:, h * HD:(h + 1) * HD])
        sh = jax.lax.dot_general(
            qh, k, (((1,), (1,)), ((), ())),
            preferred_element_type=jnp.float32)
        terms.append(jnp.maximum(sh, 0.0) * w[:, h:h + 1])
    while len(terms) > 1:
        terms = [terms[i] + terms[i + 1] for i in range(0, len(terms), 2)]
    acc = terms[0]
    row = i * BT + jax.lax.broadcasted_iota(jnp.int32, (BT, T), 0)
    col = jax.lax.broadcasted_iota(jnp.int32, (BT, T), 1)
    bits = pltpu.bitcast(acc, jnp.int32)
    key = bits ^ jax.lax.shift_right_logical(
        jax.lax.shift_right_arithmetic(bits, 31), 1)
    key = jnp.where(col > row, MASK_BASE - col, key)
    s_ref[...] = jnp.transpose(key)


def _pair_stage(n, x, ix, j, desc, tie):
    m = n // (2 * j)
    xr = x.reshape(m, 2, j, SC)
    ir = ix.reshape(m, 2, j, SC)
    a, b = xr[:, 0], xr[:, 1]
    ia, ib = ir[:, 0], ir[:, 1]
    if tie:
        gt = (a > b) | ((a == b) & (ia < ib))
    else:
        gt = a > b
    keep = gt == desc
    na = jnp.where(keep, a, b)
    nb = jnp.where(keep, b, a)
    nia = jnp.where(keep, ia, ib)
    nib = jnp.where(keep, ib, ia)
    x = jnp.stack([na, nb], axis=1).reshape(n, SC)
    ix = jnp.stack([nia, nib], axis=1).reshape(n, SC)
    return x, ix


def _roll_stage(n, x, ix, j, desc, lower, tie):
    pv = jnp.where(lower, pltpu.roll(x, n - j, 0), pltpu.roll(x, j, 0))
    pi = jnp.where(lower, pltpu.roll(ix, n - j, 0), pltpu.roll(ix, j, 0))
    if tie:
        gt = (x > pv) | ((x == pv) & (ix < pi))
    else:
        gt = x > pv
    take_self = gt == (lower == desc)
    x = jnp.where(take_self, x, pv)
    ix = jnp.where(take_self, ix, pi)
    return x, ix


TIE = True


def _make_sort_kernel(n):
    def _sort_kernel(s_ref, o_ref):
        x = s_ref[...]
        srow = jax.lax.broadcasted_iota(jnp.int32, (n, SC), 0)
        ix = srow
        k = 2
        while k <= n:
            j = k // 2
            while j >= 1:
                if j >= 8:
                    m = n // (2 * j)
                    dm = ((jax.lax.broadcasted_iota(jnp.int32, (m, 1, SC), 0)
                           * (2 * j)) & k) == 0
                    x, ix = _pair_stage(n, x, ix, j, dm, TIE)
                else:
                    desc = (srow & k) == 0
                    lower = (srow & j) == 0
                    x, ix = _roll_stage(n, x, ix, j, desc, lower, TIE)
                j //= 2
            k *= 2
        o_ref[...] = ix[:TOPK, :]
    return _sort_kernel


def kernel(x, Wqd, Wqu, Wk, Ww):
    x2 = x.reshape(T, D)
    k, w, qc = pl.pallas_call(
        _proj_kernel,
        out_shape=[
            jax.ShapeDtypeStruct((T, HD), jnp.float32),
            jax.ShapeDtypeStruct((T, H), jnp.float32),
            jax.ShapeDtypeStruct((T, D // 4), jnp.float32),
        ],
    )(x2, Wk, Ww, Wqd)

    scores_t = pl.pallas_call(
        _scores_kernel,
        grid=(T // BT,),
        in_specs=[
            pl.BlockSpec((BT, D // 4), lambda i: (i, 0)),
            pl.BlockSpec((D // 4, H * HD), lambda i: (0, 0)),
            pl.BlockSpec((T, HD), lambda i: (0, 0)),
            pl.BlockSpec((BT, H), lambda i: (i, 0)),
        ],
        out_specs=pl.BlockSpec((T, BT), lambda i: (0, i)),
        out_shape=jax.ShapeDtypeStruct((T, T), jnp.int32),
    )(qc, Wqu, k, w)

    lo_t = pl.pallas_call(
        _make_sort_kernel(TOPK),
        grid=(TOPK // SC,),
        in_specs=[pl.BlockSpec((TOPK, SC), lambda i: (0, i))],
        out_specs=pl.BlockSpec((TOPK, SC), lambda i: (0, i)),
        out_shape=jax.ShapeDtypeStruct((TOPK, TOPK), jnp.int32),
        compiler_params=pltpu.CompilerParams(
            dimension_semantics=("arbitrary",)),
    )(jax.lax.slice(scores_t, (0, 0), (TOPK, TOPK)))

    hi_t = pl.pallas_call(
        _make_sort_kernel(T),
        grid=((T - TOPK) // SC,),
        in_specs=[pl.BlockSpec((T, SC), lambda i: (0, i + TOPK // SC))],
        out_specs=pl.BlockSpec((TOPK, SC), lambda i: (0, i)),
        out_shape=jax.ShapeDtypeStruct((TOPK, T - TOPK), jnp.int32),
        compiler_params=pltpu.CompilerParams(
            dimension_semantics=("arbitrary",)),
    )(scores_t)

    idx_t = jnp.concatenate([lo_t, hi_t], axis=1)
    return jnp.transpose(idx_t).reshape(1, T, TOPK)


# speedup vs baseline: 1.2787x; 1.2787x over previous
import functools

import jax
import jax.numpy as jnp
from jax.experimental import pallas as pl
from jax.experimental.pallas import tpu as pltpu

T = 2048
D = 2048
H = 16
HD = 128
TOPK = 1024
BT = 256
SC = 128
MASK_BASE = -1900000000


def _bf(a):
    return a.astype(jnp.bfloat16)


def _proj_kernel(x_ref, wk_ref, ww_ref, wqd_ref, k_ref, w_ref, qc_ref):
    x = _bf(x_ref[...])
    k_ref[...] = jnp.dot(x, _bf(wk_ref[...]),
                         preferred_element_type=jnp.float32)
    w_ref[...] = jnp.dot(x, _bf(ww_ref[...]),
                         preferred_element_type=jnp.float32)
    qc_ref[...] = jnp.dot(x, _bf(wqd_ref[...]),
                          preferred_element_type=jnp.float32)


def _scores_kernel(qc_ref, wqu_ref, k_ref, w_ref, s_ref):
    i = pl.program_id(0)
    q = jnp.dot(_bf(qc_ref[...]), _bf(wqu_ref[...]),
                preferred_element_type=jnp.float32)
    k = _bf(k_ref[...])
    w = w_ref[...]
    terms = []
    for h in range(H):
        qh = _bf(q[:, h * HD:(h + 1) * HD])
        sh = jax.lax.dot_general(
            qh, k, (((1,), (1,)), ((), ())),
            preferred_element_type=jnp.float32)
        terms.append(jnp.maximum(sh, 0.0) * w[:, h:h + 1])
    while len(terms) > 1:
        terms = [terms[i] + terms[i + 1] for i in range(0, len(terms), 2)]
    acc = terms[0]
    row = i * BT + jax.lax.broadcasted_iota(jnp.int32, (BT, T), 0)
    col = jax.lax.broadcasted_iota(jnp.int32, (BT, T), 1)
    bits = pltpu.bitcast(acc, jnp.int32)
    key = bits ^ jax.lax.shift_right_logical(
        jax.lax.shift_right_arithmetic(bits, 31), 1)
    key = jnp.where(col > row, MASK_BASE - col, key)
    s_ref[...] = jnp.transpose(key)


def _pair_stage(n, x, ix, j, desc, tie):
    m = n // (2 * j)
    xr = x.reshape(m, 2, j, SC)
    ir = ix.reshape(m, 2, j, SC)
    a, b = xr[:, 0], xr[:, 1]
    ia, ib = ir[:, 0], ir[:, 1]
    if tie:
        gt = (a > b) | ((a == b) & (ia < ib))
    else:
        gt = a > b
    keep = gt == desc
    na = jnp.where(keep, a, b)
    nb = jnp.where(keep, b, a)
    nia = jnp.where(keep, ia, ib)
    nib = jnp.where(keep, ib, ia)
    x = jnp.stack([na, nb], axis=1).reshape(n, SC)
    ix = jnp.stack([nia, nib], axis=1).reshape(n, SC)
    return x, ix


def _roll_stage(n, x, ix, j, desc, lower, tie):
    pv = jnp.where(lower, pltpu.roll(x, n - j, 0), pltpu.roll(x, j, 0))
    pi = jnp.where(lower, pltpu.roll(ix, n - j, 0), pltpu.roll(ix, j, 0))
    if tie:
        gt = (x > pv) | ((x == pv) & (ix < pi))
    else:
        gt = x > pv
    take_self = gt == (lower == desc)
    x = jnp.where(take_self, x, pv)
    ix = jnp.where(take_self, ix, pi)
    return x, ix


TIE = False


def _make_sort_kernel(n):
    def _sort_kernel(s_ref, o_ref):
        x = s_ref[...]
        srow = jax.lax.broadcasted_iota(jnp.int32, (n, SC), 0)
        ix = srow
        k = 2
        while k <= n:
            j = k // 2
            while j >= 1:
                if j >= 8:
                    m = n // (2 * j)
                    dm = ((jax.lax.broadcasted_iota(jnp.int32, (m, 1, SC), 0)
                           * (2 * j)) & k) == 0
                    x, ix = _pair_stage(n, x, ix, j, dm, TIE)
                else:
                    desc = (srow & k) == 0
                    lower = (srow & j) == 0
                    x, ix = _roll_stage(n, x, ix, j, desc, lower, TIE)
                j //= 2
            k *= 2
        o_ref[...] = ix[:TOPK, :]
    return _sort_kernel


def kernel(x, Wqd, Wqu, Wk, Ww):
    x2 = x.reshape(T, D)
    k, w, qc = pl.pallas_call(
        _proj_kernel,
        out_shape=[
            jax.ShapeDtypeStruct((T, HD), jnp.float32),
            jax.ShapeDtypeStruct((T, H), jnp.float32),
            jax.ShapeDtypeStruct((T, D // 4), jnp.float32),
        ],
    )(x2, Wk, Ww, Wqd)

    scores_t = pl.pallas_call(
        _scores_kernel,
        grid=(T // BT,),
        in_specs=[
            pl.BlockSpec((BT, D // 4), lambda i: (i, 0)),
            pl.BlockSpec((D // 4, H * HD), lambda i: (0, 0)),
            pl.BlockSpec((T, HD), lambda i: (0, 0)),
            pl.BlockSpec((BT, H), lambda i: (i, 0)),
        ],
        out_specs=pl.BlockSpec((T, BT), lambda i: (0, i)),
        out_shape=jax.ShapeDtypeStruct((T, T), jnp.int32),
    )(qc, Wqu, k, w)

    lo_t = pl.pallas_call(
        _make_sort_kernel(TOPK),
        grid=(TOPK // SC,),
        in_specs=[pl.BlockSpec((TOPK, SC), lambda i: (0, i))],
        out_specs=pl.BlockSpec((TOPK, SC), lambda i: (0, i)),
        out_shape=jax.ShapeDtypeStruct((TOPK, TOPK), jnp.int32),
        compiler_params=pltpu.CompilerParams(
            dimension_semantics=("arbitrary",)),
    )(jax.lax.slice(scores_t, (0, 0), (TOPK, TOPK)))

    hi_t = pl.pallas_call(
        _make_sort_kernel(T),
        grid=((T - TOPK) // SC,),
        in_specs=[pl.BlockSpec((T, SC), lambda i: (0, i + TOPK // SC))],
        out_specs=pl.BlockSpec((TOPK, SC), lambda i: (0, i)),
        out_shape=jax.ShapeDtypeStruct((TOPK, T - TOPK), jnp.int32),
        compiler_params=pltpu.CompilerParams(
            dimension_semantics=("arbitrary",)),
    )(scores_t)

    idx_t = jnp.concatenate([lo_t, hi_t], axis=1)
    return jnp.transpose(idx_t).reshape(1, T, TOPK)


# speedup vs baseline: 1.2846x; 1.0046x over previous
import functools

import jax
import jax.numpy as jnp
from jax.experimental import pallas as pl
from jax.experimental.pallas import tpu as pltpu

T = 2048
D = 2048
H = 16
HD = 128
TOPK = 1024
BT = 256
SC = 128
MASK_BASE = -1900000000


def _bf(a):
    return a.astype(jnp.bfloat16)


def _proj_kernel(x_ref, wk_ref, ww_ref, wqd_ref, k_ref, w_ref, qc_ref):
    x = _bf(x_ref[...])
    k_ref[...] = jnp.dot(x, _bf(wk_ref[...]),
                         preferred_element_type=jnp.float32)
    w_ref[...] = jnp.dot(x, _bf(ww_ref[...]),
                         preferred_element_type=jnp.float32)
    qc_ref[...] = jnp.dot(x, _bf(wqd_ref[...]),
                          preferred_element_type=jnp.float32)


def _scores_kernel(qc_ref, wqu_ref, k_ref, w_ref, s_ref):
    i = pl.program_id(0)
    q = jnp.dot(_bf(qc_ref[...]), _bf(wqu_ref[...]),
                preferred_element_type=jnp.float32)
    k = _bf(k_ref[...])
    w = w_ref[...]
    terms = []
    for h in range(H):
        qh = _bf(q[:, h * HD:(h + 1) * HD])
        sh = jax.lax.dot_general(
            qh, k, (((1,), (1,)), ((), ())),
            preferred_element_type=jnp.float32)
        terms.append(jnp.maximum(sh, 0.0) * w[:, h:h + 1])
    while len(terms) > 1:
        terms = [terms[i] + terms[i + 1] for i in range(0, len(terms), 2)]
    acc = terms[0]
    row = i * BT + jax.lax.broadcasted_iota(jnp.int32, (BT, T), 0)
    col = jax.lax.broadcasted_iota(jnp.int32, (BT, T), 1)
    bits = pltpu.bitcast(acc, jnp.int32)
    key = bits ^ jax.lax.shift_right_logical(
        jax.lax.shift_right_arithmetic(bits, 31), 1)
    key = jnp.where(col > row, MASK_BASE - col, key)
    s_ref[...] = jnp.transpose(key)


def _pair_stage(n, x, ix, j, desc, tie):
    m = n // (2 * j)
    xr = x.reshape(m, 2, j, SC)
    ir = ix.reshape(m, 2, j, SC)
    a, b = xr[:, 0], xr[:, 1]
    ia, ib = ir[:, 0], ir[:, 1]
    if tie:
        gt = (a > b) | ((a == b) & (ia < ib))
    else:
        gt = a > b
    keep = gt == desc
    na = jnp.where(keep, a, b)
    nb = jnp.where(keep, b, a)
    nia = jnp.where(keep, ia, ib)
    nib = jnp.where(keep, ib, ia)
    x = jnp.stack([na, nb], axis=1).reshape(n, SC)
    ix = jnp.stack([nia, nib], axis=1).reshape(n, SC)
    return x, ix


def _roll_stage(n, x, ix, j, desc, lower, tie):
    pv = jnp.where(lower, pltpu.roll(x, n - j, 0), pltpu.roll(x, j, 0))
    pi = jnp.where(lower, pltpu.roll(ix, n - j, 0), pltpu.roll(ix, j, 0))
    if tie:
        gt = (x > pv) | ((x == pv) & (ix < pi))
    else:
        gt = x > pv
    take_self = gt == (lower == desc)
    x = jnp.where(take_self, x, pv)
    ix = jnp.where(take_self, ix, pi)
    return x, ix


TIE = False


def _run_pass(n, x, ix, srow, k, j_hi):
    j = j_hi
    while j >= 1:
        if j >= 8:
            m = n // (2 * j)
            dm = ((jax.lax.broadcasted_iota(jnp.int32, (m, 1, SC), 0)
                   * (2 * j)) & k) == 0
            x, ix = _pair_stage(n, x, ix, j, dm, TIE)
        else:
            desc = (srow & k) == 0
            lower = (srow & j) == 0
            x, ix = _roll_stage(n, x, ix, j, desc, lower, TIE)
        j //= 2
    return x, ix


def _make_sort_kernel(n):
    def _sort_kernel(s_ref, o_ref):
        x = s_ref[...]
        srow = jax.lax.broadcasted_iota(jnp.int32, (n, SC), 0)
        ix = srow
        k = 2
        while k < n:
            x, ix = _run_pass(n, x, ix, srow, k, k // 2)
            k *= 2
        if n == TOPK:
            x, ix = _run_pass(n, x, ix, srow, n, n // 2)
            o_ref[...] = ix
        else:
            half = n // 2
            a, b = x[:half], x[half:]
            ia, ib = ix[:half], ix[half:]
            if TIE:
                gt = (a > b) | ((a == b) & (ia < ib))
            else:
                gt = a > b
            x = jnp.where(gt, a, b)
            ix = jnp.where(gt, ia, ib)
            srow2 = jax.lax.broadcasted_iota(jnp.int32, (half, SC), 0)
            x, ix = _run_pass(half, x, ix, srow2, n, half // 2)
            o_ref[...] = ix[:TOPK, :]
    return _sort_kernel


def kernel(x, Wqd, Wqu, Wk, Ww):
    x2 = x.reshape(T, D)
    k, w, qc = pl.pallas_call(
        _proj_kernel,
        out_shape=[
            jax.ShapeDtypeStruct((T, HD), jnp.float32),
            jax.ShapeDtypeStruct((T, H), jnp.float32),
            jax.ShapeDtypeStruct((T, D // 4), jnp.float32),
        ],
    )(x2, Wk, Ww, Wqd)

    scores_t = pl.pallas_call(
        _scores_kernel,
        grid=(T // BT,),
        in_specs=[
            pl.BlockSpec((BT, D // 4), lambda i: (i, 0)),
            pl.BlockSpec((D // 4, H * HD), lambda i: (0, 0)),
            pl.BlockSpec((T, HD), lambda i: (0, 0)),
            pl.BlockSpec((BT, H), lambda i: (i, 0)),
        ],
        out_specs=pl.BlockSpec((T, BT), lambda i: (0, i)),
        out_shape=jax.ShapeDtypeStruct((T, T), jnp.int32),
    )(qc, Wqu, k, w)

    lo_t = pl.pallas_call(
        _make_sort_kernel(TOPK),
        grid=(TOPK // SC,),
        in_specs=[pl.BlockSpec((TOPK, SC), lambda i: (0, i))],
        out_specs=pl.BlockSpec((TOPK, SC), lambda i: (0, i)),
        out_shape=jax.ShapeDtypeStruct((TOPK, TOPK), jnp.int32),
        compiler_params=pltpu.CompilerParams(
            dimension_semantics=("arbitrary",)),
    )(jax.lax.slice(scores_t, (0, 0), (TOPK, TOPK)))

    hi_t = pl.pallas_call(
        _make_sort_kernel(T),
        grid=((T - TOPK) // SC,),
        in_specs=[pl.BlockSpec((T, SC), lambda i: (0, i + TOPK // SC))],
        out_specs=pl.BlockSpec((TOPK, SC), lambda i: (0, i)),
        out_shape=jax.ShapeDtypeStruct((TOPK, T - TOPK), jnp.int32),
        compiler_params=pltpu.CompilerParams(
            dimension_semantics=("arbitrary",)),
    )(scores_t)

    idx_t = jnp.concatenate([lo_t, hi_t], axis=1)
    return jnp.transpose(idx_t).reshape(1, T, TOPK)
